# baseline (speedup 1.0000x reference)
import jax
import jax.numpy as jnp
from jax import lax
from jax.experimental import pallas as pl
from jax.experimental.pallas import tpu as pltpu
from jax.experimental.pallas import tpu_sc as plsc

B = 64
D = 64
L = 16
NSUB = 16
B_PER = B // NSUB
NDG = D // L

_LN2 = 0.6931471805599453


def _ln16(x):
    bits = lax.bitcast_convert_type(x, jnp.int32)
    e = lax.shift_right_arithmetic(bits, 23) - 127
    m = lax.bitcast_convert_type(
        lax.bitwise_or(lax.bitwise_and(bits, jnp.int32(0x7FFFFF)),
                       jnp.int32(0x3F800000)),
        jnp.float32)
    s = (m - 1.0) / (m + 1.0)
    s2 = s * s
    p = 2.0 * s * (1.0 + s2 * (1.0 / 3.0 + s2 * (0.2 + s2 * (1.0 / 7.0 + s2 * (1.0 / 9.0)))))
    return e.astype(jnp.float32) * _LN2 + p


def _body(cw_hbm, xw_hbm, cemb_hbm, xemb_hbm, out_hbm,
          cidx_v, xidx_v, c_v, x_v, part_v, all_v, out_v, shared, sem_c, sem_x):
    sid = lax.axis_index("s")
    cid = lax.axis_index("c")

    pltpu.sync_copy(cw_hbm, cidx_v)
    pltpu.sync_copy(xw_hbm, xidx_v)
    cp_c = pltpu.async_copy(cemb_hbm.at[cidx_v], c_v, sem_c)
    cp_x = pltpu.async_copy(xemb_hbm.at[xidx_v], x_v, sem_x)
    cp_c.wait()
    cp_x.wait()

    b0 = sid * B_PER

    def kg_step(kg, accs):
        accs = list(accs)
        cvecs = [c_v[b0 + bl, pl.ds(kg * L, L)] for bl in range(B_PER)]
        for j in range(L):
            k = kg * L + j
            xrow = [x_v[k, pl.ds(L * dg, L)] for dg in range(NDG)]
            for bl in range(B_PER):
                sb = jnp.full((L,), cvecs[bl][j], jnp.float32)
                for dg in range(NDG):
                    accs[bl * NDG + dg] = accs[bl * NDG + dg] + sb * xrow[dg]
        return tuple(accs)

    zero = jnp.zeros((L,), jnp.float32)
    accs = lax.fori_loop(0, B // L, kg_step,
                         tuple(zero for _ in range(B_PER * NDG)))

    sumexp = [zero] * NDG
    sum_t = zero
    for bl in range(B_PER):
        for dg in range(NDG):
            t = accs[bl * NDG + dg]
            sumexp[dg] = sumexp[dg] + jnp.exp(t)
            sum_t = sum_t + t

    for dg in range(NDG):
        part_v[dg, :] = sumexp[dg]
    part_v[NDG, :] = sum_t

    pltpu.sync_copy(part_v, shared.at[sid])
    plsc.subcore_barrier()

    @pl.when(jnp.logical_and(sid == 0, cid == 0))
    def _():
        pltpu.sync_copy(shared, all_v)
        st = jnp.zeros((L,), jnp.float32)
        bv = jnp.zeros((L,), jnp.float32)
        for dg in range(NDG):
            se = jnp.zeros((L,), jnp.float32)
            for i in range(NSUB):
                se = se + all_v[i, dg, :]
            bv = bv + _ln16(se)
        for i in range(NSUB):
            st = st + all_v[i, NDG, :]
        bv_tot = bv[0]
        st_tot = st[0]
        for i in range(1, L):
            bv_tot = bv_tot + bv[i]
            st_tot = st_tot + st[i]
        loss = bv_tot * (1.0 / D) - st_tot * (1.0 / (D * B))
        out_v[...] = jnp.full((L,), loss, jnp.float32)
        pltpu.sync_copy(out_v, out_hbm)


_sc_loss = pl.kernel(
    _body,
    out_type=jax.ShapeDtypeStruct((L,), jnp.float32),
    mesh=plsc.VectorSubcoreMesh(core_axis_name="c", subcore_axis_name="s"),
    compiler_params=pltpu.CompilerParams(use_tc_tiling_on_sc=False),
    scratch_types=[
        pltpu.VMEM((B,), jnp.int32),
        pltpu.VMEM((B,), jnp.int32),
        pltpu.VMEM((B, D), jnp.float32),
        pltpu.VMEM((B, D), jnp.float32),
        pltpu.VMEM((NDG + 4, L), jnp.float32),
        pltpu.VMEM((NSUB, NDG + 4, L), jnp.float32),
        pltpu.VMEM((L,), jnp.float32),
        pltpu.VMEM_SHARED((NSUB, NDG + 4, L), jnp.float32),
        pltpu.SemaphoreType.DMA,
        pltpu.SemaphoreType.DMA,
    ],
)


def kernel(center_word, context_word, center_emb, context_emb):
    cw = center_word.astype(jnp.int32)
    xw = context_word.astype(jnp.int32)
    out = _sc_loss(cw, xw, center_emb, context_emb)
    return out[0]
